# Initial kernel scaffold; baseline (speedup 1.0000x reference)
import jax
import jax.numpy as jnp
from jax.experimental import pallas as pl


def kernel(A, B, index):
    raise NotImplementedError("write your pallas kernel here")


# speedup vs baseline: 37.1510x; 37.1510x over previous
import functools

import jax
import jax.numpy as jnp
from jax import lax
from jax.experimental import pallas as pl
from jax.experimental.pallas import tpu as pltpu
from jax.experimental.pallas import tpu_sc as plsc

_LANES = 16


def _matmul_body(a_ref, b_ref, s_ref):
    s_ref[...] = jnp.dot(
        a_ref[0], b_ref[0],
        preferred_element_type=jnp.float32,
        precision=jax.lax.Precision.HIGHEST,
    )[None]


def _scores(A, B, mt=512):
    BH, M, D = A.shape
    N = B.shape[2]
    return pl.pallas_call(
        _matmul_body,
        grid=(BH, M // mt),
        in_specs=[
            pl.BlockSpec((1, mt, D), lambda b, m: (b, m, 0)),
            pl.BlockSpec((1, D, N), lambda b, m: (b, 0, 0)),
        ],
        out_specs=pl.BlockSpec((1, mt, N), lambda b, m: (b, m, 0)),
        out_shape=jax.ShapeDtypeStruct((BH, M, N), jnp.float32),
    )(A, B)


def _gather_body(rows_total, N, group, nnz, n_workers,
                 s_hbm, idx_hbm, out_hbm, s_v, idx_v, out_v):
    nc = 2
    wid = lax.axis_index("s") * nc + lax.axis_index("c")
    rows_per = rows_total // n_workers
    n_groups = rows_per // group

    def body(g, carry):
        base = wid * rows_per + g * group
        pltpu.sync_copy(s_hbm.at[pl.ds(base * N, group * N)], s_v)
        pltpu.sync_copy(idx_hbm.at[pl.ds(base * nnz, group * nnz)], idx_v)
        for r in range(group):
            for c in range(nnz // _LANES):
                ids = idx_v[pl.ds(r * nnz + c * _LANES, _LANES)] + r * N
                out_v[pl.ds(r * nnz + c * _LANES, _LANES)] = plsc.load_gather(
                    s_v, [ids])
        pltpu.sync_copy(out_v, out_hbm.at[pl.ds(base * nnz, group * nnz)])
        return carry

    lax.fori_loop(0, n_groups, body, 0)


def _gather(S2, idx2, group=8):
    rows_total, N = S2.shape
    nnz = idx2.shape[1]
    n_workers = 32
    mesh = plsc.VectorSubcoreMesh(core_axis_name="c", subcore_axis_name="s")
    body = functools.partial(_gather_body, rows_total, N, group, nnz, n_workers)
    return pl.kernel(
        body,
        out_type=jax.ShapeDtypeStruct((rows_total * nnz,), jnp.float32),
        mesh=mesh,
        compiler_params=pltpu.CompilerParams(needs_layout_passes=False),
        scratch_types=[
            pltpu.VMEM((group * N,), jnp.float32),
            pltpu.VMEM((group * nnz,), jnp.int32),
            pltpu.VMEM((group * nnz,), jnp.float32),
        ],
    )(S2.reshape(-1), idx2.reshape(-1))


def kernel(A, B, index):
    BH, M, D = A.shape
    N = B.shape[2]
    nnz = index.shape[2]
    S = _scores(A, B)
    C1 = _gather(S.reshape(BH * M, N), index.reshape(BH * M, nnz))
    return C1.reshape(BH, M, nnz)


# speedup vs baseline: 52.0642x; 1.4014x over previous
import functools

import jax
import jax.numpy as jnp
from jax import lax
from jax.experimental import pallas as pl
from jax.experimental.pallas import tpu as pltpu
from jax.experimental.pallas import tpu_sc as plsc

_LANES = 16


def _matmul_body(a_ref, b_ref, s_ref):
    s_ref[...] = jnp.dot(
        a_ref[0], b_ref[0],
        preferred_element_type=jnp.float32,
        precision=jax.lax.Precision.HIGHEST,
    )[None]


def _scores(A, B, mt=512):
    BH, M, D = A.shape
    N = B.shape[2]
    return pl.pallas_call(
        _matmul_body,
        grid=(BH, M // mt),
        in_specs=[
            pl.BlockSpec((1, mt, D), lambda b, m: (b, m, 0)),
            pl.BlockSpec((1, D, N), lambda b, m: (b, 0, 0)),
        ],
        out_specs=pl.BlockSpec((1, mt, N), lambda b, m: (b, m, 0)),
        out_shape=jax.ShapeDtypeStruct((BH, M, N), jnp.float32),
    )(A, B)


def _gather_body(M, N, group, nnz,
                 s_hbm, idx_hbm, out_hbm, s_v, idx_v, out_v):
    nc = 2
    wid = lax.axis_index("s") * nc + lax.axis_index("c")
    n_groups = M // group

    def body(g, carry):
        m0 = g * group
        pltpu.sync_copy(s_hbm.at[wid, pl.ds(m0, group)], s_v)
        pltpu.sync_copy(idx_hbm.at[wid, pl.ds(m0, group)], idx_v)
        for r in range(group):
            rvec = jnp.full((_LANES,), r, jnp.int32)
            for c in range(nnz // _LANES):
                ids = idx_v[r, pl.ds(c * _LANES, _LANES)]
                out_v[r, pl.ds(c * _LANES, _LANES)] = plsc.load_gather(
                    s_v, [rvec, ids])
        pltpu.sync_copy(out_v, out_hbm.at[wid, pl.ds(m0, group)])
        return carry

    lax.fori_loop(0, n_groups, body, 0)


def _gather(S, index, group=8):
    BH, M, N = S.shape
    nnz = index.shape[2]
    mesh = plsc.VectorSubcoreMesh(core_axis_name="c", subcore_axis_name="s")
    body = functools.partial(_gather_body, M, N, group, nnz)
    return pl.kernel(
        body,
        out_type=jax.ShapeDtypeStruct((BH, M, nnz), jnp.float32),
        mesh=mesh,
        compiler_params=pltpu.CompilerParams(needs_layout_passes=False),
        scratch_types=[
            pltpu.VMEM((group, N), jnp.float32),
            pltpu.VMEM((group, nnz), jnp.int32),
            pltpu.VMEM((group, nnz), jnp.float32),
        ],
    )(S, index)


def kernel(A, B, index):
    S = _scores(A, B)
    return _gather(S, index)
